# Initial kernel scaffold; baseline (speedup 1.0000x reference)
import jax
import jax.numpy as jnp
from jax.experimental import pallas as pl


def kernel(input, lengths, weight):
    raise NotImplementedError("write your pallas kernel here")


# speedup vs baseline: 5.1435x; 5.1435x over previous
import functools

import jax
import jax.numpy as jnp
from jax import lax
from jax.experimental import pallas as pl
from jax.experimental.pallas import tpu as pltpu
from jax.experimental.pallas import tpu_sc as plsc

_NUM_EMB = 4200
_EMB_DIM = 128
_PAD = 1
_BSZ = 16
_SEQ = 4096
_HALF = _SEQ // 2
_ZROWS = 2048
_ZTILE = _ZROWS // 16

_SIZES = (2048, 1024, 512, 256, 128, 64, 32, 16, 8, 4, 2, 1)


def _body(lengths_hbm, weight_hbm, out_hbm, wslice, zshared, ztile, len_v):
    cid = lax.axis_index("c")
    sid = lax.axis_index("s")

    zeros16 = jnp.zeros((16,), jnp.float32)

    def _zero_row(r, carry):
        for kk in range(_EMB_DIM // 16):
            ztile[r, pl.ds(kk * 16, 16)] = zeros16
        return carry

    lax.fori_loop(0, _ZTILE, _zero_row, 0)

    pltpu.sync_copy(ztile, zshared.at[pl.ds(sid * _ZTILE, _ZTILE), :])
    stripe = _SEQ // 16
    pltpu.sync_copy(
        weight_hbm.at[pl.ds(_PAD + 1 + sid * stripe, stripe), :],
        wslice.at[pl.ds(sid * stripe, stripe), :],
    )
    pltpu.sync_copy(lengths_hbm, len_v)
    plsc.subcore_barrier()

    b = sid
    bvec = jnp.broadcast_to(b, (16,)).astype(jnp.int32)
    length = plsc.load_gather(len_v, [bvec])[0]

    lo = cid * _HALF
    cnt = jnp.clip(length - lo, 0, _HALF)
    zcnt = _HALF - cnt

    off = lo
    for size in _SIZES:
        take = jnp.bitwise_and(cnt, size)
        cur = off

        @pl.when(take > 0)
        def _copy(cur=cur, size=size):
            pltpu.sync_copy(
                wslice.at[pl.ds(cur, size), :],
                out_hbm.at[b, pl.ds(cur, size), :],
            )

        off = off + take

    for size in _SIZES:
        take = jnp.bitwise_and(zcnt, size)
        cur = off

        @pl.when(take > 0)
        def _fill(cur=cur, size=size):
            pltpu.sync_copy(
                zshared.at[pl.ds(0, size), :],
                out_hbm.at[b, pl.ds(cur, size), :],
            )

        off = off + take


@jax.jit
def _positional_embedding(lengths, weight):
    mesh = plsc.VectorSubcoreMesh(
        core_axis_name="c", subcore_axis_name="s", num_cores=2, num_subcores=16
    )
    return pl.kernel(
        _body,
        out_type=jax.ShapeDtypeStruct((_BSZ, _SEQ, _EMB_DIM), jnp.float32),
        mesh=mesh,
        compiler_params=pltpu.CompilerParams(
            use_tc_tiling_on_sc=False, needs_layout_passes=False
        ),
        scratch_types=[
            pltpu.VMEM_SHARED((_SEQ, _EMB_DIM), jnp.float32),
            pltpu.VMEM_SHARED((_ZROWS, _EMB_DIM), jnp.float32),
            pltpu.VMEM((_ZTILE, _EMB_DIM), jnp.float32),
            pltpu.VMEM((16,), jnp.int32),
        ],
    )(lengths, weight)


def kernel(input, lengths, weight):
    del input
    return _positional_embedding(lengths, weight)


# speedup vs baseline: 5.9402x; 1.1549x over previous
import functools

import jax
import jax.numpy as jnp
from jax import lax
from jax.experimental import pallas as pl
from jax.experimental.pallas import tpu as pltpu
from jax.experimental.pallas import tpu_sc as plsc

_NUM_EMB = 4200
_EMB_DIM = 128
_PAD = 1
_BSZ = 16
_SEQ = 4096
_HALF = _SEQ // 2
_ZROWS = 2048
_ZTILE = _ZROWS // 16

_SIZES = (2048, 1024, 512, 256, 128, 64, 32, 16, 8, 4, 2, 1)


def _body(lengths_hbm, weight_hbm, out_hbm, wslice, zshared, ztile, len_v,
          setup_sem, main_sem):
    cid = lax.axis_index("c")
    sid = lax.axis_index("s")

    stripe = _SEQ // 16
    pltpu.async_copy(
        weight_hbm.at[pl.ds(_PAD + 1 + sid * stripe, stripe), :],
        wslice.at[pl.ds(sid * stripe, stripe), :],
        setup_sem,
    )
    pltpu.async_copy(lengths_hbm, len_v, setup_sem)

    zeros16 = jnp.zeros((16,), jnp.float32)

    def _zero_row(r, carry):
        for kk in range(_EMB_DIM // 16):
            ztile[r, pl.ds(kk * 16, 16)] = zeros16
        return carry

    lax.fori_loop(0, _ZTILE, _zero_row, 0)
    pltpu.sync_copy(ztile, zshared.at[pl.ds(sid * _ZTILE, _ZTILE), :])

    pltpu.make_async_copy(
        weight_hbm.at[pl.ds(0, stripe), :],
        wslice.at[pl.ds(sid * stripe, stripe), :],
        setup_sem,
    ).wait()
    pltpu.make_async_copy(lengths_hbm, len_v, setup_sem).wait()
    plsc.subcore_barrier()

    b = sid
    bvec = jnp.broadcast_to(b, (16,)).astype(jnp.int32)
    length = plsc.load_gather(len_v, [bvec])[0]

    lo = cid * _HALF
    cnt = jnp.clip(length - lo, 0, _HALF)
    zcnt = _HALF - cnt

    off = lo
    for size in _SIZES:
        take = jnp.bitwise_and(cnt, size)
        cur = off

        @pl.when(take > 0)
        def _copy(cur=cur, size=size):
            pltpu.async_copy(
                wslice.at[pl.ds(cur, size), :],
                out_hbm.at[b, pl.ds(cur, size), :],
                main_sem,
            )

        off = off + take

    for size in _SIZES:
        take = jnp.bitwise_and(zcnt, size)
        cur = off

        @pl.when(take > 0)
        def _fill(cur=cur, size=size):
            pltpu.async_copy(
                zshared.at[pl.ds(0, size), :],
                out_hbm.at[b, pl.ds(cur, size), :],
                main_sem,
            )

        off = off + take

    pltpu.make_async_copy(
        out_hbm.at[b, pl.ds(lo, _HALF), :],
        wslice.at[pl.ds(0, _HALF), :],
        main_sem,
    ).wait()


@jax.jit
def _positional_embedding(lengths, weight):
    mesh = plsc.VectorSubcoreMesh(
        core_axis_name="c", subcore_axis_name="s", num_cores=2, num_subcores=16
    )
    return pl.kernel(
        _body,
        out_type=jax.ShapeDtypeStruct((_BSZ, _SEQ, _EMB_DIM), jnp.float32),
        mesh=mesh,
        compiler_params=pltpu.CompilerParams(
            use_tc_tiling_on_sc=False, needs_layout_passes=False
        ),
        scratch_types=[
            pltpu.VMEM_SHARED((_SEQ, _EMB_DIM), jnp.float32),
            pltpu.VMEM_SHARED((_ZROWS, _EMB_DIM), jnp.float32),
            pltpu.VMEM((_ZTILE, _EMB_DIM), jnp.float32),
            pltpu.VMEM((16,), jnp.int32),
            pltpu.SemaphoreType.DMA,
            pltpu.SemaphoreType.DMA,
        ],
    )(lengths, weight)


def kernel(input, lengths, weight):
    del input
    return _positional_embedding(lengths, weight)
